# Initial kernel scaffold; baseline (speedup 1.0000x reference)
import jax
import jax.numpy as jnp
from jax.experimental import pallas as pl


def kernel(x, edge_index, W_rnn, h0, a_prelu, W_dec):
    raise NotImplementedError("write your pallas kernel here")


# speedup vs baseline: 4.8703x; 4.8703x over previous
import functools

import jax
import jax.numpy as jnp
from jax import lax
from jax.experimental import pallas as pl
from jax.experimental.pallas import tpu as pltpu
from jax.experimental.pallas import tpu_sc as plsc

_N = 10000
_D = 128
_E = 320000
_C = 32
_K1 = 17
_KP = 32

_NC = 2
_NS = 16
_NW = _NC * _NS
_CH = 128
_CPW = -(-_E // (_NW * _CH))
_EP = _NW * _CH * _CPW
_RPS = 8 * (-(-(_N + _NS) // (8 * _NS)))
_NP = _RPS * _NS
_BN = 2000


def _sc_scatter_body(x_hbm, srcb_hbm, dstb_hbm, zeros_hbm, out_hbm,
                     src_v, dst_v, rows_v, acc_sh, sem):
    c = lax.axis_index("c")
    s = lax.axis_index("s")
    wid = s * _NC + c

    pltpu.sync_copy(zeros_hbm, acc_sh.at[pl.ds(s * _RPS, _RPS)])
    plsc.subcore_barrier()

    pltpu.sync_copy(srcb_hbm.at[wid], src_v)
    pltpu.sync_copy(dstb_hbm.at[wid], dst_v)

    def chunk(j, carry):
        pltpu.async_copy(x_hbm.at[src_v.at[j]], rows_v, sem).wait()
        pltpu.sync_copy(rows_v, acc_sh.at[dst_v.at[j]], add=True)
        return carry

    lax.fori_loop(0, _CPW, chunk, 0)
    plsc.subcore_barrier()

    pltpu.sync_copy(acc_sh.at[pl.ds(s * _RPS, _RPS)],
                    out_hbm.at[c, pl.ds(s * _RPS, _RPS)])


def _make_sc_scatter():
    return functools.partial(
        pl.kernel,
        mesh=plsc.VectorSubcoreMesh(core_axis_name="c", subcore_axis_name="s",
                                    num_cores=_NC, num_subcores=_NS),
        out_type=jax.ShapeDtypeStruct((_NC, _NP, _D), jnp.float32),
        scratch_types=[
            pltpu.VMEM((_CPW, _CH), jnp.int32),
            pltpu.VMEM((_CPW, _CH), jnp.int32),
            pltpu.VMEM((_CH, _D), jnp.float32),
            pltpu.VMEM_SHARED((_NP, _D), jnp.float32),
            pltpu.SemaphoreType.DMA,
        ],
        compiler_params=pltpu.CompilerParams(use_tc_tiling_on_sc=False),
    )(_sc_scatter_body)


def _tc_finish_kernel(part_ref, wrnn_ref, h0_ref, a_ref, wdec_ref,
                      scores_ref, topics_ref):
    a = a_ref[0]
    wrnn_t = wrnn_ref[...].T
    row = lax.broadcasted_iota(jnp.int32, (_C, _C), 0)

    def step(i, carry):
        h, H = carry
        h = jnp.dot(h, wrnn_t, preferred_element_type=jnp.float32)
        h = jnp.where(h >= 0, h, a * h)
        H = jnp.where(row == i, h, H)
        return h, H

    _, H = lax.fori_loop(0, _K1, step,
                         (h0_ref[...], jnp.zeros((_C, _C), jnp.float32)))
    weights = jnp.dot(H, wdec_ref[...].T, preferred_element_type=jnp.float32)

    xa = part_ref[0] + part_ref[1]
    s2 = jnp.dot(xa, weights.T, preferred_element_type=jnp.float32)
    col = lax.broadcasted_iota(jnp.int32, (_BN, _KP), 1)
    sm = jnp.where(col < _K1, s2, -jnp.inf)
    m = jnp.max(sm, axis=1, keepdims=True)
    t = jnp.min(jnp.where(sm >= m, col, _KP), axis=1)
    scores_ref[...] = s2
    topics_ref[...] = jnp.broadcast_to(t[:, None], (_BN, _KP))


def kernel(x, edge_index, W_rnn, h0, a_prelu, W_dec):
    src = edge_index[0]
    dst = edge_index[1]
    pad = _EP - _E
    srcb = jnp.concatenate([src, jnp.zeros((pad,), jnp.int32)]
                           ).reshape(_NW, _CPW, _CH)
    dstb = jnp.concatenate([dst, jnp.full((pad,), _N, jnp.int32)]
                           ).reshape(_NW, _CPW, _CH)
    zeros = jnp.zeros((_RPS, _D), jnp.float32)

    part = _make_sc_scatter()(x, srcb, dstb, zeros)

    scores_pad, topics2d = pl.pallas_call(
        _tc_finish_kernel,
        grid=(_N // _BN,),
        in_specs=[
            pl.BlockSpec((_NC, _BN, _D), lambda i: (0, i, 0)),
            pl.BlockSpec((_C, _C), lambda i: (0, 0)),
            pl.BlockSpec((1, _C), lambda i: (0, 0)),
            pl.BlockSpec(memory_space=pltpu.SMEM),
            pl.BlockSpec((_D, _C), lambda i: (0, 0)),
        ],
        out_specs=[pl.BlockSpec((_BN, _KP), lambda i: (i, 0)),
                   pl.BlockSpec((_BN, _KP), lambda i: (i, 0))],
        out_shape=[jax.ShapeDtypeStruct((_N, _KP), jnp.float32),
                   jax.ShapeDtypeStruct((_N, _KP), jnp.int32)],
    )(part, W_rnn, h0.reshape(1, _C), a_prelu.reshape(1), W_dec)

    return scores_pad[:, :_K1], topics2d[:, 0]


# speedup vs baseline: 5.7485x; 1.1803x over previous
import functools

import jax
import jax.numpy as jnp
from jax import lax
from jax.experimental import pallas as pl
from jax.experimental.pallas import tpu as pltpu
from jax.experimental.pallas import tpu_sc as plsc

_N = 10000
_D = 128
_E = 320000
_C = 32
_K1 = 17
_KP = 32

_NC = 2
_NS = 16
_NW = _NC * _NS
_CH = 128
_CPW = -(-_E // (_NW * _CH))
_EP = _NW * _CH * _CPW
_RPS = 8 * (-(-(_N + _NS) // (8 * _NS)))
_NP = _RPS * _NS
_BN = 2000
_NBUF = 2
_NBI = 4


def _sc_scatter_body(x_hbm, srcb_hbm, dstb_hbm, zeros_hbm, out_hbm,
                     src_v, dst_v, rows_v, acc_sh, sem_i, sem_r):
    c = lax.axis_index("c")
    s = lax.axis_index("s")
    wid = s * _NC + c

    pltpu.sync_copy(zeros_hbm, acc_sh.at[pl.ds(s * _RPS, _RPS)])
    plsc.subcore_barrier()

    def start_idx(j):
        b = lax.rem(j, _NBI)
        pltpu.async_copy(srcb_hbm.at[wid, j], src_v.at[b], sem_i)
        pltpu.async_copy(dstb_hbm.at[wid, j], dst_v.at[b], sem_i)

    def wait_idx(j):
        b = lax.rem(j, _NBI)
        pltpu.make_async_copy(srcb_hbm.at[wid, j], src_v.at[b], sem_i).wait()
        pltpu.make_async_copy(dstb_hbm.at[wid, j], dst_v.at[b], sem_i).wait()

    def start_rows(j):
        pltpu.async_copy(x_hbm.at[src_v.at[lax.rem(j, _NBI)]],
                         rows_v.at[lax.rem(j, _NBUF)], sem_r)

    start_idx(0)
    start_idx(1)
    wait_idx(0)
    start_rows(0)

    def chunk(j, carry):
        @pl.when(j + 2 < _CPW)
        def _():
            start_idx(j + 2)

        @pl.when(j + 1 < _CPW)
        def _():
            wait_idx(j + 1)
            start_rows(j + 1)

        b = lax.rem(j, _NBUF)
        bi = lax.rem(j, _NBI)
        pltpu.make_async_copy(x_hbm.at[src_v.at[bi]], rows_v.at[b],
                              sem_r).wait()
        pltpu.sync_copy(rows_v.at[b], acc_sh.at[dst_v.at[bi]], add=True)
        return carry

    lax.fori_loop(0, _CPW, chunk, 0)
    plsc.subcore_barrier()

    pltpu.sync_copy(acc_sh.at[pl.ds(s * _RPS, _RPS)],
                    out_hbm.at[c, pl.ds(s * _RPS, _RPS)])


def _make_sc_scatter():
    return functools.partial(
        pl.kernel,
        mesh=plsc.VectorSubcoreMesh(core_axis_name="c", subcore_axis_name="s",
                                    num_cores=_NC, num_subcores=_NS),
        out_type=jax.ShapeDtypeStruct((_NC, _NP, _D), jnp.float32),
        scratch_types=[
            pltpu.VMEM((_NBI, _CH), jnp.int32),
            pltpu.VMEM((_NBI, _CH), jnp.int32),
            pltpu.VMEM((_NBUF, _CH, _D), jnp.float32),
            pltpu.VMEM_SHARED((_NP, _D), jnp.float32),
            pltpu.SemaphoreType.DMA,
            pltpu.SemaphoreType.DMA,
        ],
        compiler_params=pltpu.CompilerParams(use_tc_tiling_on_sc=False),
    )(_sc_scatter_body)


def _tc_finish_kernel(part_ref, wrnn_ref, h0_ref, a_ref, wdec_ref,
                      scores_ref, topics_ref):
    a = a_ref[0]
    wrnn_t = wrnn_ref[...].T
    row = lax.broadcasted_iota(jnp.int32, (_C, _C), 0)

    def step(i, carry):
        h, H = carry
        h = jnp.dot(h, wrnn_t, preferred_element_type=jnp.float32)
        h = jnp.where(h >= 0, h, a * h)
        H = jnp.where(row == i, h, H)
        return h, H

    _, H = lax.fori_loop(0, _K1, step,
                         (h0_ref[...], jnp.zeros((_C, _C), jnp.float32)))
    weights = jnp.dot(H, wdec_ref[...].T, preferred_element_type=jnp.float32)

    xa = part_ref[0] + part_ref[1]
    s2 = jnp.dot(xa, weights.T, preferred_element_type=jnp.float32)
    col = lax.broadcasted_iota(jnp.int32, (_BN, _KP), 1)
    sm = jnp.where(col < _K1, s2, -jnp.inf)
    m = jnp.max(sm, axis=1, keepdims=True)
    t = jnp.min(jnp.where(sm >= m, col, _KP), axis=1)
    scores_ref[...] = s2
    topics_ref[...] = jnp.broadcast_to(t[:, None], (_BN, _KP))


def kernel(x, edge_index, W_rnn, h0, a_prelu, W_dec):
    src = edge_index[0]
    dst = edge_index[1]
    pad = _EP - _E
    srcb = jnp.concatenate([src, jnp.zeros((pad,), jnp.int32)]
                           ).reshape(_NW, _CPW, _CH)
    dstb = jnp.concatenate([dst, jnp.full((pad,), _N, jnp.int32)]
                           ).reshape(_NW, _CPW, _CH)
    zeros = jnp.zeros((_RPS, _D), jnp.float32)

    part = _make_sc_scatter()(x, srcb, dstb, zeros)

    scores_pad, topics2d = pl.pallas_call(
        _tc_finish_kernel,
        grid=(_N // _BN,),
        in_specs=[
            pl.BlockSpec((_NC, _BN, _D), lambda i: (0, i, 0)),
            pl.BlockSpec((_C, _C), lambda i: (0, 0)),
            pl.BlockSpec((1, _C), lambda i: (0, 0)),
            pl.BlockSpec(memory_space=pltpu.SMEM),
            pl.BlockSpec((_D, _C), lambda i: (0, 0)),
        ],
        out_specs=[pl.BlockSpec((_BN, _KP), lambda i: (i, 0)),
                   pl.BlockSpec((_BN, _KP), lambda i: (i, 0))],
        out_shape=[jax.ShapeDtypeStruct((_N, _KP), jnp.float32),
                   jax.ShapeDtypeStruct((_N, _KP), jnp.int32)],
    )(part, W_rnn, h0.reshape(1, _C), a_prelu.reshape(1), W_dec)

    return scores_pad[:, :_K1], topics2d[:, 0]


# speedup vs baseline: 7.2798x; 1.2664x over previous
import functools

import jax
import jax.numpy as jnp
from jax import lax
from jax.experimental import pallas as pl
from jax.experimental.pallas import tpu as pltpu
from jax.experimental.pallas import tpu_sc as plsc

_N = 10000
_D = 128
_E = 320000
_C = 32
_K1 = 17
_KP = 32

_NC = 2
_NS = 16
_NW = _NC * _NS
_CH = 128
_KC0 = 45
_KC1 = 112
_NCHUNK = _NS * (_KC0 + _KC1)
_EP = _NCHUNK * _CH
_RPS = 8 * (-(-(_N + _NS) // (8 * _NS)))
_NP = _RPS * _NS
_BN = 2000
_NBUF = 2
_NBI = 4


def _sc_scatter_body(x_hbm, srcb_hbm, dstb_hbm, zeros_hbm, out_hbm,
                     src_v, dst_v, rows_v, acc_sh, sem_i, sem_r):
    c = lax.axis_index("c")
    s = lax.axis_index("s")
    nk = lax.select(c == 0, _KC0, _KC1)
    base = c * (_NS * _KC0) + s * nk

    pltpu.sync_copy(zeros_hbm, acc_sh.at[pl.ds(s * _RPS, _RPS)])
    plsc.subcore_barrier()

    def start_idx(j):
        b = lax.rem(j, _NBI)
        pltpu.async_copy(srcb_hbm.at[base + j], src_v.at[b], sem_i)
        pltpu.async_copy(dstb_hbm.at[base + j], dst_v.at[b], sem_i)

    def wait_idx(j):
        b = lax.rem(j, _NBI)
        pltpu.make_async_copy(srcb_hbm.at[base + j], src_v.at[b], sem_i).wait()
        pltpu.make_async_copy(dstb_hbm.at[base + j], dst_v.at[b], sem_i).wait()

    def start_rows(j):
        pltpu.async_copy(x_hbm.at[src_v.at[lax.rem(j, _NBI)]],
                         rows_v.at[lax.rem(j, _NBUF)], sem_r)

    start_idx(0)
    start_idx(1)
    wait_idx(0)
    start_rows(0)

    def chunk(j, carry):
        @pl.when(j + 2 < nk)
        def _():
            start_idx(j + 2)

        @pl.when(j + 1 < nk)
        def _():
            wait_idx(j + 1)
            start_rows(j + 1)

        b = lax.rem(j, _NBUF)
        bi = lax.rem(j, _NBI)
        pltpu.make_async_copy(x_hbm.at[src_v.at[bi]], rows_v.at[b],
                              sem_r).wait()
        pltpu.sync_copy(rows_v.at[b], acc_sh.at[dst_v.at[bi]], add=True)
        return carry

    lax.fori_loop(0, nk, chunk, 0)
    plsc.subcore_barrier()

    pltpu.sync_copy(acc_sh.at[pl.ds(s * _RPS, _RPS)],
                    out_hbm.at[c, pl.ds(s * _RPS, _RPS)])


def _make_sc_scatter():
    return functools.partial(
        pl.kernel,
        mesh=plsc.VectorSubcoreMesh(core_axis_name="c", subcore_axis_name="s",
                                    num_cores=_NC, num_subcores=_NS),
        out_type=jax.ShapeDtypeStruct((_NC, _NP, _D), jnp.float32),
        scratch_types=[
            pltpu.VMEM((_NBI, _CH), jnp.int32),
            pltpu.VMEM((_NBI, _CH), jnp.int32),
            pltpu.VMEM((_NBUF, _CH, _D), jnp.float32),
            pltpu.VMEM_SHARED((_NP, _D), jnp.float32),
            pltpu.SemaphoreType.DMA,
            pltpu.SemaphoreType.DMA,
        ],
        compiler_params=pltpu.CompilerParams(use_tc_tiling_on_sc=False),
    )(_sc_scatter_body)


def _tc_finish_kernel(part_ref, wrnn_ref, h0_ref, a_ref, wdec_ref,
                      scores_ref, topics_ref):
    a = a_ref[0]
    wrnn_t = wrnn_ref[...].T
    row = lax.broadcasted_iota(jnp.int32, (_C, _C), 0)

    def step(i, carry):
        h, H = carry
        h = jnp.dot(h, wrnn_t, preferred_element_type=jnp.float32)
        h = jnp.where(h >= 0, h, a * h)
        H = jnp.where(row == i, h, H)
        return h, H

    _, H = lax.fori_loop(0, _K1, step,
                         (h0_ref[...], jnp.zeros((_C, _C), jnp.float32)))
    weights = jnp.dot(H, wdec_ref[...].T, preferred_element_type=jnp.float32)

    xa = part_ref[0] + part_ref[1]
    s2 = jnp.dot(xa, weights.T, preferred_element_type=jnp.float32)
    col = lax.broadcasted_iota(jnp.int32, (_BN, _KP), 1)
    sm = jnp.where(col < _K1, s2, -jnp.inf)
    m = jnp.max(sm, axis=1, keepdims=True)
    t = jnp.min(jnp.where(sm >= m, col, _KP), axis=1)
    scores_ref[...] = s2
    topics_ref[...] = jnp.broadcast_to(t[:, None], (_BN, _KP))


def kernel(x, edge_index, W_rnn, h0, a_prelu, W_dec):
    src = edge_index[0]
    dst = edge_index[1]
    pad = _EP - _E
    srcb = jnp.concatenate([src, jnp.zeros((pad,), jnp.int32)]
                           ).reshape(_NCHUNK, _CH)
    dstb = jnp.concatenate([dst, jnp.full((pad,), _N, jnp.int32)]
                           ).reshape(_NCHUNK, _CH)
    zeros = jnp.zeros((_RPS, _D), jnp.float32)

    part = _make_sc_scatter()(x, srcb, dstb, zeros)

    scores_pad, topics2d = pl.pallas_call(
        _tc_finish_kernel,
        grid=(_N // _BN,),
        in_specs=[
            pl.BlockSpec((_NC, _BN, _D), lambda i: (0, i, 0)),
            pl.BlockSpec((_C, _C), lambda i: (0, 0)),
            pl.BlockSpec((1, _C), lambda i: (0, 0)),
            pl.BlockSpec(memory_space=pltpu.SMEM),
            pl.BlockSpec((_D, _C), lambda i: (0, 0)),
        ],
        out_specs=[pl.BlockSpec((_BN, _KP), lambda i: (i, 0)),
                   pl.BlockSpec((_BN, _KP), lambda i: (i, 0))],
        out_shape=[jax.ShapeDtypeStruct((_N, _KP), jnp.float32),
                   jax.ShapeDtypeStruct((_N, _KP), jnp.int32)],
    )(part, W_rnn, h0.reshape(1, _C), a_prelu.reshape(1), W_dec)

    return scores_pad[:, :_K1], topics2d[:, 0]


# speedup vs baseline: 9.1493x; 1.2568x over previous
import functools

import jax
import jax.numpy as jnp
from jax import lax
from jax.experimental import pallas as pl
from jax.experimental.pallas import tpu as pltpu
from jax.experimental.pallas import tpu_sc as plsc

_N = 10000
_D = 128
_E = 320000
_C = 32
_K1 = 17
_KP = 32

_NC = 2
_NS = 16
_NW = _NC * _NS
_CH = 128
_KC0 = 112
_KC1 = 45
_NCHUNK = _NS * (_KC0 + _KC1)
_EP = _NCHUNK * _CH
_RPS = 8 * (-(-(_N + _NS) // (8 * _NS)))
_NP = _RPS * _NS
_BN = 2000
_NBUF = 2
_NBI = 4


def _sc_scatter_body(x_hbm, srcb_hbm, dstb_hbm, zeros_hbm, out_hbm,
                     src_v, dst_v, rows_v, acc_sh, sem_i, sem_r):
    c = lax.axis_index("c")
    s = lax.axis_index("s")
    nk = lax.select(c == 0, _KC0, _KC1)
    base = c * (_NS * _KC0) + s * nk

    pltpu.sync_copy(zeros_hbm, acc_sh.at[pl.ds(s * _RPS, _RPS)])
    plsc.subcore_barrier()

    def start_idx(j):
        b = lax.rem(j, _NBI)
        pltpu.async_copy(srcb_hbm.at[base + j], src_v.at[b], sem_i)
        pltpu.async_copy(dstb_hbm.at[base + j], dst_v.at[b], sem_i)

    def wait_idx(j):
        b = lax.rem(j, _NBI)
        pltpu.make_async_copy(srcb_hbm.at[base + j], src_v.at[b], sem_i).wait()
        pltpu.make_async_copy(dstb_hbm.at[base + j], dst_v.at[b], sem_i).wait()

    def start_rows(j):
        pltpu.async_copy(x_hbm.at[src_v.at[lax.rem(j, _NBI)]],
                         rows_v.at[lax.rem(j, _NBUF)], sem_r)

    start_idx(0)
    start_idx(1)
    wait_idx(0)
    start_rows(0)

    def chunk(j, carry):
        @pl.when(j + 2 < nk)
        def _():
            start_idx(j + 2)

        @pl.when(j + 1 < nk)
        def _():
            wait_idx(j + 1)
            start_rows(j + 1)

        b = lax.rem(j, _NBUF)
        bi = lax.rem(j, _NBI)
        pltpu.make_async_copy(x_hbm.at[src_v.at[bi]], rows_v.at[b],
                              sem_r).wait()
        pltpu.sync_copy(rows_v.at[b], acc_sh.at[dst_v.at[bi]], add=True)
        return carry

    lax.fori_loop(0, nk, chunk, 0)
    plsc.subcore_barrier()

    pltpu.sync_copy(acc_sh.at[pl.ds(s * _RPS, _RPS)],
                    out_hbm.at[c, pl.ds(s * _RPS, _RPS)])


def _make_sc_scatter():
    return functools.partial(
        pl.kernel,
        mesh=plsc.VectorSubcoreMesh(core_axis_name="c", subcore_axis_name="s",
                                    num_cores=_NC, num_subcores=_NS),
        out_type=jax.ShapeDtypeStruct((_NC, _NP, _D), jnp.float32),
        scratch_types=[
            pltpu.VMEM((_NBI, _CH), jnp.int32),
            pltpu.VMEM((_NBI, _CH), jnp.int32),
            pltpu.VMEM((_NBUF, _CH, _D), jnp.float32),
            pltpu.VMEM_SHARED((_NP, _D), jnp.float32),
            pltpu.SemaphoreType.DMA,
            pltpu.SemaphoreType.DMA,
        ],
        compiler_params=pltpu.CompilerParams(use_tc_tiling_on_sc=False),
    )(_sc_scatter_body)


def _tc_finish_kernel(part_ref, wrnn_ref, h0_ref, a_ref, wdec_ref,
                      scores_ref, topics_ref):
    a = a_ref[0]
    wrnn_t = wrnn_ref[...].T
    row = lax.broadcasted_iota(jnp.int32, (_C, _C), 0)

    def step(i, carry):
        h, H = carry
        h = jnp.dot(h, wrnn_t, preferred_element_type=jnp.float32)
        h = jnp.where(h >= 0, h, a * h)
        H = jnp.where(row == i, h, H)
        return h, H

    _, H = lax.fori_loop(0, _K1, step,
                         (h0_ref[...], jnp.zeros((_C, _C), jnp.float32)))
    weights = jnp.dot(H, wdec_ref[...].T, preferred_element_type=jnp.float32)

    xa = part_ref[0] + part_ref[1]
    s2 = jnp.dot(xa, weights.T, preferred_element_type=jnp.float32)
    col = lax.broadcasted_iota(jnp.int32, (_BN, _KP), 1)
    sm = jnp.where(col < _K1, s2, -jnp.inf)
    m = jnp.max(sm, axis=1, keepdims=True)
    t = jnp.min(jnp.where(sm >= m, col, _KP), axis=1)
    scores_ref[...] = s2
    topics_ref[...] = jnp.broadcast_to(t[:, None], (_BN, _KP))


def kernel(x, edge_index, W_rnn, h0, a_prelu, W_dec):
    src = edge_index[0]
    dst = edge_index[1]
    pad = _EP - _E
    srcb = jnp.concatenate([src, jnp.zeros((pad,), jnp.int32)]
                           ).reshape(_NCHUNK, _CH)
    dstb = jnp.concatenate([dst, jnp.full((pad,), _N, jnp.int32)]
                           ).reshape(_NCHUNK, _CH)
    zeros = jnp.zeros((_RPS, _D), jnp.float32)

    part = _make_sc_scatter()(x, srcb, dstb, zeros)

    scores_pad, topics2d = pl.pallas_call(
        _tc_finish_kernel,
        grid=(_N // _BN,),
        in_specs=[
            pl.BlockSpec((_NC, _BN, _D), lambda i: (0, i, 0)),
            pl.BlockSpec((_C, _C), lambda i: (0, 0)),
            pl.BlockSpec((1, _C), lambda i: (0, 0)),
            pl.BlockSpec(memory_space=pltpu.SMEM),
            pl.BlockSpec((_D, _C), lambda i: (0, 0)),
        ],
        out_specs=[pl.BlockSpec((_BN, _KP), lambda i: (i, 0)),
                   pl.BlockSpec((_BN, _KP), lambda i: (i, 0))],
        out_shape=[jax.ShapeDtypeStruct((_N, _KP), jnp.float32),
                   jax.ShapeDtypeStruct((_N, _KP), jnp.int32)],
    )(part, W_rnn, h0.reshape(1, _C), a_prelu.reshape(1), W_dec)

    return scores_pad[:, :_K1], topics2d[:, 0]


# speedup vs baseline: 9.1510x; 1.0002x over previous
import functools

import jax
import jax.numpy as jnp
from jax import lax
from jax.experimental import pallas as pl
from jax.experimental.pallas import tpu as pltpu
from jax.experimental.pallas import tpu_sc as plsc

_N = 10000
_D = 128
_E = 320000
_C = 32
_K1 = 17
_KP = 32

_NC = 2
_NS = 16
_NW = _NC * _NS
_CH = 128
_KC0 = 112
_KC1 = 45
_NCHUNK = _NS * (_KC0 + _KC1)
_EP = _NCHUNK * _CH
_RPS = 8 * (-(-(_N + _NS) // (8 * _NS)))
_NP = _RPS * _NS
_BN = 2000
_NBUF = 2
_NBI = 4


def _sc_scatter_body(x_hbm, srcb_hbm, dstb_hbm, zeros_hbm, out_hbm,
                     src_v, dst_v, rows_v, acc_sh, sem_i, sem_r, sem_s):
    c = lax.axis_index("c")
    s = lax.axis_index("s")
    nk = lax.select(c == 0, _KC0, _KC1)
    base = c * (_NS * _KC0) + s * nk

    pltpu.sync_copy(zeros_hbm, acc_sh.at[pl.ds(s * _RPS, _RPS)])
    plsc.subcore_barrier()

    def start_idx(j):
        b = lax.rem(j, _NBI)
        pltpu.async_copy(srcb_hbm.at[base + j], src_v.at[b], sem_i)
        pltpu.async_copy(dstb_hbm.at[base + j], dst_v.at[b], sem_i)

    def wait_idx(j):
        b = lax.rem(j, _NBI)
        pltpu.make_async_copy(srcb_hbm.at[base + j], src_v.at[b], sem_i).wait()
        pltpu.make_async_copy(dstb_hbm.at[base + j], dst_v.at[b], sem_i).wait()

    def start_rows(j):
        pltpu.async_copy(x_hbm.at[src_v.at[lax.rem(j, _NBI)]],
                         rows_v.at[lax.rem(j, _NBUF)], sem_r)

    def scatter_desc(j):
        return pltpu.make_async_copy(
            rows_v.at[lax.rem(j, _NBUF)],
            acc_sh.at[dst_v.at[lax.rem(j, _NBI)]], sem_s)

    start_idx(0)
    start_idx(1)
    wait_idx(0)
    start_rows(0)

    def chunk(j, carry):
        @pl.when(j + 2 < nk)
        def _():
            start_idx(j + 2)

        @pl.when(j + 1 < nk)
        def _():
            wait_idx(j + 1)

            @pl.when(j >= 1)
            def _():
                scatter_desc(j - 1).wait()

            start_rows(j + 1)

        b = lax.rem(j, _NBUF)
        bi = lax.rem(j, _NBI)
        pltpu.make_async_copy(x_hbm.at[src_v.at[bi]], rows_v.at[b],
                              sem_r).wait()
        pltpu.async_copy(rows_v.at[b], acc_sh.at[dst_v.at[bi]], sem_s,
                         add=True)
        return carry

    lax.fori_loop(0, nk, chunk, 0)
    scatter_desc(nk - 2).wait()
    scatter_desc(nk - 1).wait()
    plsc.subcore_barrier()

    pltpu.sync_copy(acc_sh.at[pl.ds(s * _RPS, _RPS)],
                    out_hbm.at[c, pl.ds(s * _RPS, _RPS)])


def _make_sc_scatter():
    return functools.partial(
        pl.kernel,
        mesh=plsc.VectorSubcoreMesh(core_axis_name="c", subcore_axis_name="s",
                                    num_cores=_NC, num_subcores=_NS),
        out_type=jax.ShapeDtypeStruct((_NC, _NP, _D), jnp.float32),
        scratch_types=[
            pltpu.VMEM((_NBI, _CH), jnp.int32),
            pltpu.VMEM((_NBI, _CH), jnp.int32),
            pltpu.VMEM((_NBUF, _CH, _D), jnp.float32),
            pltpu.VMEM_SHARED((_NP, _D), jnp.float32),
            pltpu.SemaphoreType.DMA,
            pltpu.SemaphoreType.DMA,
            pltpu.SemaphoreType.DMA,
        ],
        compiler_params=pltpu.CompilerParams(use_tc_tiling_on_sc=False),
    )(_sc_scatter_body)


def _tc_finish_kernel(part_ref, wrnn_ref, h0_ref, a_ref, wdec_ref,
                      scores_ref, topics_ref):
    a = a_ref[0]
    wrnn_t = wrnn_ref[...].T
    row = lax.broadcasted_iota(jnp.int32, (_C, _C), 0)

    def step(i, carry):
        h, H = carry
        h = jnp.dot(h, wrnn_t, preferred_element_type=jnp.float32)
        h = jnp.where(h >= 0, h, a * h)
        H = jnp.where(row == i, h, H)
        return h, H

    _, H = lax.fori_loop(0, _K1, step,
                         (h0_ref[...], jnp.zeros((_C, _C), jnp.float32)))
    weights = jnp.dot(H, wdec_ref[...].T, preferred_element_type=jnp.float32)

    xa = part_ref[0] + part_ref[1]
    s2 = jnp.dot(xa, weights.T, preferred_element_type=jnp.float32)
    col = lax.broadcasted_iota(jnp.int32, (_BN, _KP), 1)
    sm = jnp.where(col < _K1, s2, -jnp.inf)
    m = jnp.max(sm, axis=1, keepdims=True)
    t = jnp.min(jnp.where(sm >= m, col, _KP), axis=1)
    scores_ref[...] = s2
    topics_ref[...] = jnp.broadcast_to(t[:, None], (_BN, _KP))


def kernel(x, edge_index, W_rnn, h0, a_prelu, W_dec):
    src = edge_index[0]
    dst = edge_index[1]
    pad = _EP - _E
    srcb = jnp.concatenate([src, jnp.zeros((pad,), jnp.int32)]
                           ).reshape(_NCHUNK, _CH)
    dstb = jnp.concatenate([dst, jnp.full((pad,), _N, jnp.int32)]
                           ).reshape(_NCHUNK, _CH)
    zeros = jnp.zeros((_RPS, _D), jnp.float32)

    part = _make_sc_scatter()(x, srcb, dstb, zeros)

    scores_pad, topics2d = pl.pallas_call(
        _tc_finish_kernel,
        grid=(_N // _BN,),
        in_specs=[
            pl.BlockSpec((_NC, _BN, _D), lambda i: (0, i, 0)),
            pl.BlockSpec((_C, _C), lambda i: (0, 0)),
            pl.BlockSpec((1, _C), lambda i: (0, 0)),
            pl.BlockSpec(memory_space=pltpu.SMEM),
            pl.BlockSpec((_D, _C), lambda i: (0, 0)),
        ],
        out_specs=[pl.BlockSpec((_BN, _KP), lambda i: (i, 0)),
                   pl.BlockSpec((_BN, _KP), lambda i: (i, 0))],
        out_shape=[jax.ShapeDtypeStruct((_N, _KP), jnp.float32),
                   jax.ShapeDtypeStruct((_N, _KP), jnp.int32)],
    )(part, W_rnn, h0.reshape(1, _C), a_prelu.reshape(1), W_dec)

    return scores_pad[:, :_K1], topics2d[:, 0]


# speedup vs baseline: 10.2547x; 1.1206x over previous
import functools

import jax
import jax.numpy as jnp
from jax import lax
from jax.experimental import pallas as pl
from jax.experimental.pallas import tpu as pltpu
from jax.experimental.pallas import tpu_sc as plsc

_N = 10000
_D = 128
_E = 320000
_C = 32
_K1 = 17
_KP = 32

_NC = 2
_NS = 16
_NW = _NC * _NS
_CH = 128
_NCHUNK = _E // _CH
_T0 = 1754
_T1 = _NCHUNK - _T0
_RPS = 8 * (-(-(_N + _NS) // (8 * _NS)))
_NP = _RPS * _NS
_BN = 2000
_NBUF = 2
_NBI = 4


def _sc_scatter_body(x_hbm, eib_hbm, zeros_hbm, out_hbm,
                     src_v, dst_v, rows_v, acc_sh, sem_i, sem_r, sem_s):
    c = lax.axis_index("c")
    s = lax.axis_index("s")
    t = lax.select(c == 0, _T0, _T1)
    offs = lax.select(c == 0, 0, _T0)
    base = offs + lax.div(s * t, _NS)
    nk = offs + lax.div((s + 1) * t, _NS) - base

    def start_idx(j):
        b = lax.rem(j, _NBI)
        pltpu.async_copy(eib_hbm.at[0, base + j], src_v.at[b], sem_i)
        pltpu.async_copy(eib_hbm.at[1, base + j], dst_v.at[b], sem_i)

    def wait_idx(j):
        b = lax.rem(j, _NBI)
        pltpu.make_async_copy(eib_hbm.at[0, base + j], src_v.at[b], sem_i).wait()
        pltpu.make_async_copy(eib_hbm.at[1, base + j], dst_v.at[b], sem_i).wait()

    def start_rows(j):
        pltpu.async_copy(x_hbm.at[src_v.at[lax.rem(j, _NBI)]],
                         rows_v.at[lax.rem(j, _NBUF)], sem_r)

    def scatter_desc(j):
        return pltpu.make_async_copy(
            rows_v.at[lax.rem(j, _NBUF)],
            acc_sh.at[dst_v.at[lax.rem(j, _NBI)]], sem_s)

    start_idx(0)
    start_idx(1)
    pltpu.sync_copy(zeros_hbm, acc_sh.at[pl.ds(s * _RPS, _RPS)])
    wait_idx(0)
    start_rows(0)
    plsc.subcore_barrier()

    def chunk(j, carry):
        @pl.when(j + 2 < nk)
        def _():
            start_idx(j + 2)

        @pl.when(j + 1 < nk)
        def _():
            wait_idx(j + 1)

            @pl.when(j >= 1)
            def _():
                scatter_desc(j - 1).wait()

            start_rows(j + 1)

        b = lax.rem(j, _NBUF)
        bi = lax.rem(j, _NBI)
        pltpu.make_async_copy(x_hbm.at[src_v.at[bi]], rows_v.at[b],
                              sem_r).wait()
        pltpu.async_copy(rows_v.at[b], acc_sh.at[dst_v.at[bi]], sem_s,
                         add=True)
        return carry

    lax.fori_loop(0, nk, chunk, 0)
    scatter_desc(nk - 2).wait()
    scatter_desc(nk - 1).wait()
    plsc.subcore_barrier()

    pltpu.sync_copy(acc_sh.at[pl.ds(s * _RPS, _RPS)],
                    out_hbm.at[c, pl.ds(s * _RPS, _RPS)])


def _make_sc_scatter():
    return functools.partial(
        pl.kernel,
        mesh=plsc.VectorSubcoreMesh(core_axis_name="c", subcore_axis_name="s",
                                    num_cores=_NC, num_subcores=_NS),
        out_type=jax.ShapeDtypeStruct((_NC, _NP, _D), jnp.float32),
        scratch_types=[
            pltpu.VMEM((_NBI, _CH), jnp.int32),
            pltpu.VMEM((_NBI, _CH), jnp.int32),
            pltpu.VMEM((_NBUF, _CH, _D), jnp.float32),
            pltpu.VMEM_SHARED((_NP, _D), jnp.float32),
            pltpu.SemaphoreType.DMA,
            pltpu.SemaphoreType.DMA,
            pltpu.SemaphoreType.DMA,
        ],
        compiler_params=pltpu.CompilerParams(use_tc_tiling_on_sc=False),
    )(_sc_scatter_body)


def _tc_finish_kernel(part_ref, wrnn_ref, h0_ref, a_ref, wdec_ref,
                      scores_ref, topics_ref):
    a = a_ref[0]
    wrnn_t = wrnn_ref[...].T
    row = lax.broadcasted_iota(jnp.int32, (_C, _C), 0)

    def step(i, carry):
        h, H = carry
        h = jnp.dot(h, wrnn_t, preferred_element_type=jnp.float32)
        h = jnp.where(h >= 0, h, a * h)
        H = jnp.where(row == i, h, H)
        return h, H

    _, H = lax.fori_loop(0, _K1, step,
                         (h0_ref[...], jnp.zeros((_C, _C), jnp.float32)))
    weights = jnp.dot(H, wdec_ref[...].T, preferred_element_type=jnp.float32)

    xa = part_ref[0] + part_ref[1]
    s2 = jnp.dot(xa, weights.T, preferred_element_type=jnp.float32)
    col = lax.broadcasted_iota(jnp.int32, (_BN, _KP), 1)
    sm = jnp.where(col < _K1, s2, -jnp.inf)
    m = jnp.max(sm, axis=1, keepdims=True)
    t = jnp.min(jnp.where(sm >= m, col, _KP), axis=1)
    scores_ref[...] = s2[:, :_K1]
    topics_ref[...] = t[:, None]


def kernel(x, edge_index, W_rnn, h0, a_prelu, W_dec):
    eib = edge_index.reshape(2, _NCHUNK, _CH)
    zeros = jnp.zeros((_RPS, _D), jnp.float32)

    part = _make_sc_scatter()(x, eib, zeros)

    scores, topics2d = pl.pallas_call(
        _tc_finish_kernel,
        grid=(_N // _BN,),
        in_specs=[
            pl.BlockSpec((_NC, _BN, _D), lambda i: (0, i, 0)),
            pl.BlockSpec((_C, _C), lambda i: (0, 0)),
            pl.BlockSpec((1, _C), lambda i: (0, 0)),
            pl.BlockSpec(memory_space=pltpu.SMEM),
            pl.BlockSpec((_D, _C), lambda i: (0, 0)),
        ],
        out_specs=[pl.BlockSpec((_BN, _K1), lambda i: (i, 0)),
                   pl.BlockSpec((_BN, 1), lambda i: (i, 0))],
        out_shape=[jax.ShapeDtypeStruct((_N, _K1), jnp.float32),
                   jax.ShapeDtypeStruct((_N, 1), jnp.int32)],
    )(part, W_rnn, h0.reshape(1, _C), a_prelu.reshape(1), W_dec)

    return scores, topics2d.reshape(_N)


# speedup vs baseline: 13.1704x; 1.2843x over previous
import functools

import jax
import jax.numpy as jnp
from jax import lax
from jax.experimental import pallas as pl
from jax.experimental.pallas import tpu as pltpu
from jax.experimental.pallas import tpu_sc as plsc

_N = 10000
_D = 128
_E = 320000
_C = 32
_K1 = 17
_KP = 32

_NC = 2
_NS = 16
_NW = _NC * _NS
_CH = 128
_NCHUNK = _E // _CH
_T0 = 1250
_T1 = _NCHUNK - _T0
_RPS = 8 * (-(-(_N + _NS) // (8 * _NS)))
_NP = _RPS * _NS
_BN = 10000
_NBUF = 2
_NBI = 4


def _sc_scatter_body(x_hbm, eib_hbm, zeros_hbm, out_hbm,
                     src_v, dst_v, rows_v, acc_sh, sem_i, sem_r, sem_s):
    c = lax.axis_index("c")
    s = lax.axis_index("s")
    t = lax.select(c == 0, _T0, _T1)
    offs = lax.select(c == 0, 0, _T0)
    base = offs + lax.div(s * t, _NS)
    nk = offs + lax.div((s + 1) * t, _NS) - base

    def start_idx(j):
        b = lax.rem(j, _NBI)
        pltpu.async_copy(eib_hbm.at[0, base + j], src_v.at[b], sem_i)
        pltpu.async_copy(eib_hbm.at[1, base + j], dst_v.at[b], sem_i)

    def wait_idx(j):
        b = lax.rem(j, _NBI)
        pltpu.make_async_copy(eib_hbm.at[0, base + j], src_v.at[b], sem_i).wait()
        pltpu.make_async_copy(eib_hbm.at[1, base + j], dst_v.at[b], sem_i).wait()

    def start_rows(j):
        pltpu.async_copy(x_hbm.at[src_v.at[lax.rem(j, _NBI)]],
                         rows_v.at[lax.rem(j, _NBUF)], sem_r)

    def scatter_desc(j):
        return pltpu.make_async_copy(
            rows_v.at[lax.rem(j, _NBUF)],
            acc_sh.at[dst_v.at[lax.rem(j, _NBI)]], sem_s)

    start_idx(0)
    start_idx(1)
    pltpu.sync_copy(zeros_hbm, acc_sh.at[pl.ds(s * _RPS, _RPS)])
    wait_idx(0)
    start_rows(0)
    plsc.subcore_barrier()

    def chunk(j, carry):
        @pl.when(j + 2 < nk)
        def _():
            start_idx(j + 2)

        @pl.when(j + 1 < nk)
        def _():
            wait_idx(j + 1)

            @pl.when(j >= 1)
            def _():
                scatter_desc(j - 1).wait()

            start_rows(j + 1)

        b = lax.rem(j, _NBUF)
        bi = lax.rem(j, _NBI)
        pltpu.make_async_copy(x_hbm.at[src_v.at[bi]], rows_v.at[b],
                              sem_r).wait()
        pltpu.async_copy(rows_v.at[b], acc_sh.at[dst_v.at[bi]], sem_s,
                         add=True)
        return carry

    lax.fori_loop(0, nk, chunk, 0)
    scatter_desc(nk - 2).wait()
    scatter_desc(nk - 1).wait()
    plsc.subcore_barrier()

    pltpu.sync_copy(acc_sh.at[pl.ds(s * _RPS, _RPS)],
                    out_hbm.at[c, pl.ds(s * _RPS, _RPS)])


def _make_sc_scatter():
    return functools.partial(
        pl.kernel,
        mesh=plsc.VectorSubcoreMesh(core_axis_name="c", subcore_axis_name="s",
                                    num_cores=_NC, num_subcores=_NS),
        out_type=jax.ShapeDtypeStruct((_NC, _NP, _D), jnp.float32),
        scratch_types=[
            pltpu.VMEM((_NBI, _CH), jnp.int32),
            pltpu.VMEM((_NBI, _CH), jnp.int32),
            pltpu.VMEM((_NBUF, _CH, _D), jnp.float32),
            pltpu.VMEM_SHARED((_NP, _D), jnp.float32),
            pltpu.SemaphoreType.DMA,
            pltpu.SemaphoreType.DMA,
            pltpu.SemaphoreType.DMA,
        ],
        compiler_params=pltpu.CompilerParams(use_tc_tiling_on_sc=False),
    )(_sc_scatter_body)


def _tc_finish_kernel(part_ref, wrnn_ref, h0_ref, a_ref, wdec_ref,
                      scores_ref, topics_ref):
    a = a_ref[0]
    wrnn_t = wrnn_ref[...].T
    row = lax.broadcasted_iota(jnp.int32, (_C, _C), 0)

    def step(i, carry):
        h, H = carry
        h = jnp.dot(h, wrnn_t, preferred_element_type=jnp.float32)
        h = jnp.where(h >= 0, h, a * h)
        H = jnp.where(row == i, h, H)
        return h, H

    _, H = lax.fori_loop(0, _K1, step,
                         (h0_ref[...], jnp.zeros((_C, _C), jnp.float32)))
    weights = jnp.dot(H, wdec_ref[...].T, preferred_element_type=jnp.float32)

    xa = part_ref[0] + part_ref[1]
    s2 = jnp.dot(xa, weights.T, preferred_element_type=jnp.float32)
    col = lax.broadcasted_iota(jnp.int32, (_BN, _KP), 1)
    sm = jnp.where(col < _K1, s2, -jnp.inf)
    m = jnp.max(sm, axis=1, keepdims=True)
    t = jnp.min(jnp.where(sm >= m, col, _KP), axis=1)
    scores_ref[...] = s2[:, :_K1]
    topics_ref[...] = t[:, None]


def kernel(x, edge_index, W_rnn, h0, a_prelu, W_dec):
    eib = edge_index.reshape(2, _NCHUNK, _CH)
    zeros = jnp.zeros((_RPS, _D), jnp.float32)

    part = _make_sc_scatter()(x, eib, zeros)

    scores, topics2d = pl.pallas_call(
        _tc_finish_kernel,
        grid=(_N // _BN,),
        in_specs=[
            pl.BlockSpec((_NC, _BN, _D), lambda i: (0, 0, 0)),
            pl.BlockSpec((_C, _C), lambda i: (0, 0)),
            pl.BlockSpec((1, _C), lambda i: (0, 0)),
            pl.BlockSpec(memory_space=pltpu.SMEM),
            pl.BlockSpec((_D, _C), lambda i: (0, 0)),
        ],
        out_specs=[pl.BlockSpec((_BN, _K1), lambda i: (0, 0)),
                   pl.BlockSpec((_BN, 1), lambda i: (0, 0))],
        out_shape=[jax.ShapeDtypeStruct((_N, _K1), jnp.float32),
                   jax.ShapeDtypeStruct((_N, 1), jnp.int32)],
    )(part, W_rnn, h0.reshape(1, _C), a_prelu.reshape(1), W_dec)

    return scores, topics2d.reshape(_N)


# speedup vs baseline: 13.5418x; 1.0282x over previous
import functools

import jax
import jax.numpy as jnp
from jax import lax
from jax.experimental import pallas as pl
from jax.experimental.pallas import tpu as pltpu
from jax.experimental.pallas import tpu_sc as plsc

_N = 10000
_D = 128
_E = 320000
_C = 32
_K1 = 17
_KP = 32

_NC = 2
_NS = 16
_NW = _NC * _NS
_CH = 128
_NCHUNK = _E // _CH
_T0 = 1250
_T1 = _NCHUNK - _T0
_RPS = 8 * (-(-(_N + _NS) // (8 * _NS)))
_NP = _RPS * _NS
_BN = 10000
_NBUF = 2
_NBI = 4


def _sc_scatter_body(x_hbm, eib_hbm, zeros_hbm, out_hbm,
                     src_v, dst_v, rows_v, acc_sh, sem_i, sem_r, sem_s):
    c = lax.axis_index("c")
    s = lax.axis_index("s")
    t = lax.select(c == 0, _T0, _T1)
    offs = lax.select(c == 0, 0, _T0)
    base = offs + lax.div(s * t, _NS)
    nk = offs + lax.div((s + 1) * t, _NS) - base

    def start_idx(j):
        b = lax.rem(j, _NBI)
        pltpu.async_copy(eib_hbm.at[0, base + j], src_v.at[b], sem_i)
        pltpu.async_copy(eib_hbm.at[1, base + j], dst_v.at[b], sem_i)

    def wait_idx(j):
        b = lax.rem(j, _NBI)
        pltpu.make_async_copy(eib_hbm.at[0, base + j], src_v.at[b], sem_i).wait()
        pltpu.make_async_copy(eib_hbm.at[1, base + j], dst_v.at[b], sem_i).wait()

    def start_rows(j):
        pltpu.async_copy(x_hbm.at[src_v.at[lax.rem(j, _NBI)]],
                         rows_v.at[lax.rem(j, _NBUF)], sem_r)

    def scatter_desc(j):
        return pltpu.make_async_copy(
            rows_v.at[lax.rem(j, _NBUF)],
            acc_sh.at[dst_v.at[lax.rem(j, _NBI)]], sem_s)

    start_idx(0)
    start_idx(1)
    pltpu.sync_copy(zeros_hbm, acc_sh.at[pl.ds(s * _RPS, _RPS)])
    wait_idx(0)
    start_rows(0)
    plsc.subcore_barrier()

    def chunk(j, carry):
        @pl.when(j + 2 < nk)
        def _():
            start_idx(j + 2)

        @pl.when(j + 1 < nk)
        def _():
            wait_idx(j + 1)

            @pl.when(j >= 1)
            def _():
                scatter_desc(j - 1).wait()

            start_rows(j + 1)

        b = lax.rem(j, _NBUF)
        bi = lax.rem(j, _NBI)
        pltpu.make_async_copy(x_hbm.at[src_v.at[bi]], rows_v.at[b],
                              sem_r).wait()
        pltpu.async_copy(rows_v.at[b], acc_sh.at[dst_v.at[bi]], sem_s,
                         add=True)
        return carry

    lax.fori_loop(0, nk, chunk, 0)
    scatter_desc(nk - 2).wait()
    scatter_desc(nk - 1).wait()
    plsc.subcore_barrier()

    pltpu.sync_copy(acc_sh.at[pl.ds(s * _RPS, _RPS)],
                    out_hbm.at[c, pl.ds(s * _RPS, _RPS)])


def _make_sc_scatter():
    return functools.partial(
        pl.kernel,
        mesh=plsc.VectorSubcoreMesh(core_axis_name="c", subcore_axis_name="s",
                                    num_cores=_NC, num_subcores=_NS),
        out_type=jax.ShapeDtypeStruct((_NC, _NP, _D), jnp.float32),
        scratch_types=[
            pltpu.VMEM((_NBI, _CH), jnp.int32),
            pltpu.VMEM((_NBI, _CH), jnp.int32),
            pltpu.VMEM((_NBUF, _CH, _D), jnp.float32),
            pltpu.VMEM_SHARED((_NP, _D), jnp.float32),
            pltpu.SemaphoreType.DMA,
            pltpu.SemaphoreType.DMA,
            pltpu.SemaphoreType.DMA,
        ],
        compiler_params=pltpu.CompilerParams(use_tc_tiling_on_sc=False),
    )(_sc_scatter_body)


def _tc_finish_kernel(part_ref, wrnn_ref, h0_ref, a_ref, wdec_ref,
                      scores_ref, topics_ref):
    a = a_ref[0]
    wrnn_t = wrnn_ref[...].T
    row = lax.broadcasted_iota(jnp.int32, (_C, _C), 0)

    def step(i, carry):
        h, H = carry
        h = jnp.dot(h, wrnn_t, preferred_element_type=jnp.float32)
        h = jnp.where(h >= 0, h, a * h)
        H = jnp.where(row == i, h, H)
        return h, H

    _, H = lax.fori_loop(0, _K1, step,
                         (h0_ref[...], jnp.zeros((_C, _C), jnp.float32)))
    weights = jnp.dot(H, wdec_ref[...].T, preferred_element_type=jnp.float32)

    xa = part_ref[0] + part_ref[1]
    s2 = jnp.dot(xa, weights.T, preferred_element_type=jnp.float32)
    col = lax.broadcasted_iota(jnp.int32, (_BN, _KP), 1)
    sm = jnp.where(col < _K1, s2, -jnp.inf)
    m = jnp.max(sm, axis=1, keepdims=True)
    t = jnp.min(jnp.where(sm >= m, col, _KP), axis=1)
    scores_ref[...] = s2.T[:_K1, :]
    topics_ref[...] = t


def kernel(x, edge_index, W_rnn, h0, a_prelu, W_dec):
    eib = edge_index.reshape(2, _NCHUNK, _CH)
    zeros = jnp.zeros((_RPS, _D), jnp.float32)

    part = _make_sc_scatter()(x, eib, zeros)

    scores, topics2d = pl.pallas_call(
        _tc_finish_kernel,
        grid=(_N // _BN,),
        in_specs=[
            pl.BlockSpec((_NC, _BN, _D), lambda i: (0, 0, 0)),
            pl.BlockSpec((_C, _C), lambda i: (0, 0)),
            pl.BlockSpec((1, _C), lambda i: (0, 0)),
            pl.BlockSpec(memory_space=pltpu.SMEM),
            pl.BlockSpec((_D, _C), lambda i: (0, 0)),
        ],
        out_specs=[pl.BlockSpec((_K1, _BN), lambda i: (0, 0)),
                   pl.BlockSpec((_BN,), lambda i: (0,))],
        out_shape=[jax.ShapeDtypeStruct((_K1, _N), jnp.float32),
                   jax.ShapeDtypeStruct((_N,), jnp.int32)],
    )(part, W_rnn, h0.reshape(1, _C), a_prelu.reshape(1), W_dec)

    return scores.T, topics2d


# speedup vs baseline: 13.6058x; 1.0047x over previous
import functools

import jax
import jax.numpy as jnp
from jax import lax
from jax.experimental import pallas as pl
from jax.experimental.pallas import tpu as pltpu
from jax.experimental.pallas import tpu_sc as plsc

_N = 10000
_D = 128
_E = 320000
_C = 32
_K1 = 17
_KP = 32

_NC = 2
_NS = 16
_NW = _NC * _NS
_CH = 128
_NCHUNK = _E // _CH
_T0 = 1250
_T1 = _NCHUNK - _T0
_RPS = 8 * (-(-(_N + _NS) // (8 * _NS)))
_NP = _RPS * _NS
_BN = 10000
_NBUF = 2
_NBI = 4


def _sc_scatter_body(x_hbm, eib_hbm, zeros_hbm, out_hbm,
                     src_v, dst_v, rows_v, acc_sh, sem_i, sem_r, sem_s):
    c = lax.axis_index("c")
    s = lax.axis_index("s")
    t = lax.select(c == 0, _T0, _T1)
    offs = lax.select(c == 0, 0, _T0)
    base = offs + lax.div(s * t, _NS)
    nk = offs + lax.div((s + 1) * t, _NS) - base

    def start_idx(j):
        b = lax.rem(j, _NBI)
        pltpu.async_copy(eib_hbm.at[0, base + j], src_v.at[b], sem_i)
        pltpu.async_copy(eib_hbm.at[1, base + j], dst_v.at[b], sem_i)

    def wait_idx(j):
        b = lax.rem(j, _NBI)
        pltpu.make_async_copy(eib_hbm.at[0, base + j], src_v.at[b], sem_i).wait()
        pltpu.make_async_copy(eib_hbm.at[1, base + j], dst_v.at[b], sem_i).wait()

    def start_rows(j):
        pltpu.async_copy(x_hbm.at[src_v.at[lax.rem(j, _NBI)]],
                         rows_v.at[lax.rem(j, _NBUF)], sem_r)

    def scatter_desc(j):
        return pltpu.make_async_copy(
            rows_v.at[lax.rem(j, _NBUF)],
            acc_sh.at[dst_v.at[lax.rem(j, _NBI)]], sem_s)

    start_idx(0)
    start_idx(1)
    pltpu.sync_copy(zeros_hbm, acc_sh.at[pl.ds(s * _RPS, _RPS)])
    wait_idx(0)
    start_rows(0)
    plsc.subcore_barrier()

    def chunk(j, carry):
        @pl.when(j + 2 < nk)
        def _():
            start_idx(j + 2)

        @pl.when(j + 1 < nk)
        def _():
            wait_idx(j + 1)

            @pl.when(j >= 1)
            def _():
                scatter_desc(j - 1).wait()

            start_rows(j + 1)

        b = lax.rem(j, _NBUF)
        bi = lax.rem(j, _NBI)
        pltpu.make_async_copy(x_hbm.at[src_v.at[bi]], rows_v.at[b],
                              sem_r).wait()
        pltpu.async_copy(rows_v.at[b], acc_sh.at[dst_v.at[bi]], sem_s,
                         add=True)
        return carry

    lax.fori_loop(0, nk, chunk, 0)
    scatter_desc(nk - 2).wait()
    scatter_desc(nk - 1).wait()
    plsc.subcore_barrier()

    pltpu.sync_copy(acc_sh.at[pl.ds(s * _RPS, _RPS)],
                    out_hbm.at[c, pl.ds(s * _RPS, _RPS)])


def _make_sc_scatter():
    return functools.partial(
        pl.kernel,
        mesh=plsc.VectorSubcoreMesh(core_axis_name="c", subcore_axis_name="s",
                                    num_cores=_NC, num_subcores=_NS),
        out_type=jax.ShapeDtypeStruct((_NC, _NP, _D), jnp.float32),
        scratch_types=[
            pltpu.VMEM((_NBI, _CH), jnp.int32),
            pltpu.VMEM((_NBI, _CH), jnp.int32),
            pltpu.VMEM((_NBUF, _CH, _D), jnp.float32),
            pltpu.VMEM_SHARED((_NP, _D), jnp.float32),
            pltpu.SemaphoreType.DMA,
            pltpu.SemaphoreType.DMA,
            pltpu.SemaphoreType.DMA,
        ],
        compiler_params=pltpu.CompilerParams(use_tc_tiling_on_sc=False),
    )(_sc_scatter_body)


def _tc_finish_kernel(part_ref, wrnn_ref, h0_ref, a_ref, wdec_ref,
                      scores_ref, topics_ref, wt_ref):
    @pl.when(pl.program_id(0) == 0)
    def _():
        a = a_ref[0]
        wrnn_t = wrnn_ref[...].T
        row = lax.broadcasted_iota(jnp.int32, (_C, _C), 0)

        def step(i, carry):
            h, H = carry
            h = jnp.dot(h, wrnn_t, preferred_element_type=jnp.float32)
            h = jnp.where(h >= 0, h, a * h)
            H = jnp.where(row == i, h, H)
            return h, H

        _, H = lax.fori_loop(0, _K1, step,
                             (h0_ref[...], jnp.zeros((_C, _C), jnp.float32)))
        wt_ref[...] = jnp.dot(H, wdec_ref[...].T,
                              preferred_element_type=jnp.float32)

    weights = wt_ref[...]
    xa = part_ref[0] + part_ref[1]
    s2 = jnp.dot(xa, weights.T, preferred_element_type=jnp.float32)
    col = lax.broadcasted_iota(jnp.int32, (_BN, _KP), 1)
    sm = jnp.where(col < _K1, s2, -jnp.inf)
    m = jnp.max(sm, axis=1, keepdims=True)
    t = jnp.min(jnp.where(sm >= m, col, _KP), axis=1)
    scores_ref[...] = s2.T[:_K1, :]
    topics_ref[...] = t


def kernel(x, edge_index, W_rnn, h0, a_prelu, W_dec):
    eib = edge_index.reshape(2, _NCHUNK, _CH)
    zeros = jnp.zeros((_RPS, _D), jnp.float32)

    part = _make_sc_scatter()(x, eib, zeros)

    scores, topics2d = pl.pallas_call(
        _tc_finish_kernel,
        grid=(_N // _BN,),
        in_specs=[
            pl.BlockSpec((_NC, _BN, _D), lambda i: (0, 0, 0)),
            pl.BlockSpec((_C, _C), lambda i: (0, 0)),
            pl.BlockSpec((1, _C), lambda i: (0, 0)),
            pl.BlockSpec(memory_space=pltpu.SMEM),
            pl.BlockSpec((_D, _C), lambda i: (0, 0)),
        ],
        out_specs=[pl.BlockSpec((_K1, _N), lambda i: (0, 0)),
                   pl.BlockSpec((_N,), lambda i: (0,))],
        out_shape=[jax.ShapeDtypeStruct((_K1, _N), jnp.float32),
                   jax.ShapeDtypeStruct((_N,), jnp.int32)],
        scratch_shapes=[pltpu.VMEM((_KP, _D), jnp.float32)],
    )(part, W_rnn, h0.reshape(1, _C), a_prelu.reshape(1), W_dec)

    return scores.T, topics2d
